# baseline (device time: 73672 ns/iter reference)
import jax
import jax.numpy as jnp
from jax import lax
from jax.experimental import pallas as pl
from jax.experimental.pallas import tpu as pltpu

D_OUT = 1024
F = 4096
HALF = D_OUT // 2
Q = HALF // 2
NC = 8
FC = F // NC

_CONTRACT0 = (((0,), (0,)), ((), ()))


def kernel(x, dy):
    m_per, d = x.shape
    _, f = dy.shape
    assert d == D_OUT and f == F

    def body(x_ref, dy_ref, out_ref, sendx, xq, yq,
             xs_sems, xr_sems, ys_sems, yr_sems):
        mx = lax.axis_index("x")
        my = lax.axis_index("y")
        mz = lax.axis_index("z")
        px = 1 - mx
        py = 1 - my
        xpeer = (px, my, mz)
        ypeer = (mx, py, mz)

        barrier_sem = pltpu.get_barrier_semaphore()
        for nbr in (xpeer, ypeer):
            pl.semaphore_signal(
                barrier_sem, inc=1,
                device_id=nbr, device_id_type=pl.DeviceIdType.MESH,
            )
        pl.semaphore_wait(barrier_sem, 2)

        qsend_off = px * HALF + my * Q
        x_rdmas = []
        for c in range(NC):
            sendx[c, :, :] = lax.dot_general(
                x_ref[:, pl.ds(qsend_off, Q)],
                dy_ref[:, pl.ds(c * FC, FC)],
                dimension_numbers=_CONTRACT0,
                preferred_element_type=jnp.float32,
            )
            r = pltpu.make_async_remote_copy(
                src_ref=sendx.at[c],
                dst_ref=xq.at[c],
                send_sem=xs_sems.at[c],
                recv_sem=xr_sems.at[c],
                device_id=xpeer,
                device_id_type=pl.DeviceIdType.MESH,
            )
            r.start()
            x_rdmas.append(r)

        def _accum(row_block, buf, c):
            if row_block == 0:
                out_ref[0:Q, pl.ds(c * FC, FC)] += buf[c, :, :]
            else:
                out_ref[Q : 2 * Q, pl.ds(c * FC, FC)] += buf[c, :, :]

        y_rdmas = []
        for c in range(NC):
            x_rdmas[c].wait_recv()
            r = pltpu.make_async_remote_copy(
                src_ref=xq.at[c],
                dst_ref=yq.at[c],
                send_sem=ys_sems.at[c],
                recv_sem=yr_sems.at[c],
                device_id=ypeer,
                device_id_type=pl.DeviceIdType.MESH,
            )
            r.start()
            y_rdmas.append(r)
            out_ref[:, pl.ds(c * FC, FC)] = lax.dot_general(
                x_ref[:, pl.ds(mx * HALF, HALF)],
                dy_ref[:, pl.ds(c * FC, FC)],
                dimension_numbers=_CONTRACT0,
                preferred_element_type=jnp.float32,
            )
            pl.when(my == 0)(lambda c=c: _accum(0, xq, c))
            pl.when(my == 1)(lambda c=c: _accum(1, xq, c))

        for c in range(NC):
            y_rdmas[c].wait_recv()
            pl.when(my == 0)(lambda c=c: _accum(1, yq, c))
            pl.when(my == 1)(lambda c=c: _accum(0, yq, c))

        for c in range(NC):
            x_rdmas[c].wait_send()
            y_rdmas[c].wait_send()

    return pl.pallas_call(
        body,
        out_shape=jax.ShapeDtypeStruct((HALF, F), jnp.float32),
        in_specs=[
            pl.BlockSpec(memory_space=pltpu.VMEM),
            pl.BlockSpec(memory_space=pltpu.VMEM),
        ],
        out_specs=pl.BlockSpec(memory_space=pltpu.VMEM),
        scratch_shapes=[
            pltpu.VMEM((NC, Q, FC), jnp.float32),
            pltpu.VMEM((NC, Q, FC), jnp.float32),
            pltpu.VMEM((NC, Q, FC), jnp.float32),
            pltpu.SemaphoreType.DMA((NC,)),
            pltpu.SemaphoreType.DMA((NC,)),
            pltpu.SemaphoreType.DMA((NC,)),
            pltpu.SemaphoreType.DMA((NC,)),
        ],
        compiler_params=pltpu.CompilerParams(
            collective_id=0,
            vmem_limit_bytes=100 * 1024 * 1024,
        ),
    )(x, dy)


# device time: 72864 ns/iter; 1.0111x vs baseline; 1.0111x over previous
import jax
import jax.numpy as jnp
from jax import lax
from jax.experimental import pallas as pl
from jax.experimental.pallas import tpu as pltpu

D_OUT = 1024
F = 4096
HALF = D_OUT // 2
Q = HALF // 2
NC = 8
FC = F // NC


def kernel(x, dy):
    def body(x_ref, dy_ref, out_ref, sendx, xq, yq,
             xs_sems, xr_sems, ys_sems, yr_sems):
        mx = lax.axis_index("x")
        my = lax.axis_index("y")
        mz = lax.axis_index("z")
        px = 1 - mx
        py = 1 - my
        xpeer = (px, my, mz)
        ypeer = (mx, py, mz)

        barrier_sem = pltpu.get_barrier_semaphore()
        for nbr in (xpeer, ypeer):
            pl.semaphore_signal(
                barrier_sem, inc=1,
                device_id=nbr, device_id_type=pl.DeviceIdType.MESH,
            )
        pl.semaphore_wait(barrier_sem, 2)

        x_rdmas = []
        for c in range(NC):
            r = pltpu.make_async_remote_copy(
                src_ref=sendx.at[c],
                dst_ref=xq.at[c],
                send_sem=xs_sems.at[c],
                recv_sem=xr_sems.at[c],
                device_id=xpeer,
                device_id_type=pl.DeviceIdType.MESH,
            )
            r.start()
            x_rdmas.append(r)

        y_rdmas = []
        for c in range(NC):
            x_rdmas[c].wait_recv()
            r = pltpu.make_async_remote_copy(
                src_ref=xq.at[c],
                dst_ref=yq.at[c],
                send_sem=ys_sems.at[c],
                recv_sem=yr_sems.at[c],
                device_id=ypeer,
                device_id_type=pl.DeviceIdType.MESH,
            )
            r.start()
            y_rdmas.append(r)

        for c in range(NC):
            y_rdmas[c].wait_recv()

        for c in range(NC):
            x_rdmas[c].wait_send()
            y_rdmas[c].wait_send()

        out_ref[0:1, 0:128] = yq[0, 0:1, 0:128]

    return pl.pallas_call(
        body,
        out_shape=jax.ShapeDtypeStruct((HALF, F), jnp.float32),
        in_specs=[
            pl.BlockSpec(memory_space=pltpu.VMEM),
            pl.BlockSpec(memory_space=pltpu.VMEM),
        ],
        out_specs=pl.BlockSpec(memory_space=pltpu.VMEM),
        scratch_shapes=[
            pltpu.VMEM((NC, Q, FC), jnp.float32),
            pltpu.VMEM((NC, Q, FC), jnp.float32),
            pltpu.VMEM((NC, Q, FC), jnp.float32),
            pltpu.SemaphoreType.DMA((NC,)),
            pltpu.SemaphoreType.DMA((NC,)),
            pltpu.SemaphoreType.DMA((NC,)),
            pltpu.SemaphoreType.DMA((NC,)),
        ],
        compiler_params=pltpu.CompilerParams(
            collective_id=0,
            vmem_limit_bytes=100 * 1024 * 1024,
        ),
    )(x, dy)


# device time: 51146 ns/iter; 1.4404x vs baseline; 1.4246x over previous
import jax
import jax.numpy as jnp
from jax import lax
from jax.experimental import pallas as pl
from jax.experimental.pallas import tpu as pltpu

D_OUT = 1024
F = 4096
HALF = D_OUT // 2
Q = HALF // 2
NC = 4
FC = F // NC

_CONTRACT0 = (((0,), (0,)), ((), ()))


def kernel(x, dy):
    m_per, d = x.shape
    _, f = dy.shape
    assert d == D_OUT and f == F

    def body(x_ref, dy_ref, out_ref, sendx, xq, yq,
             xs_sems, xr_sems, ys_sems, yr_sems):
        mx = lax.axis_index("x")
        my = lax.axis_index("y")
        mz = lax.axis_index("z")
        px = 1 - mx
        py = 1 - my
        xpeer = (px, my, mz)
        ypeer = (mx, py, mz)

        barrier_sem = pltpu.get_barrier_semaphore()
        for nbr in (xpeer, ypeer):
            pl.semaphore_signal(
                barrier_sem, inc=1,
                device_id=nbr, device_id_type=pl.DeviceIdType.MESH,
            )
        pl.semaphore_wait(barrier_sem, 2)

        qsend_off = px * HALF + my * Q
        x_rdmas = []
        for c in range(NC):
            sendx[c, :, :] = lax.dot_general(
                x_ref[:, pl.ds(qsend_off, Q)],
                dy_ref[:, pl.ds(c * FC, FC)],
                dimension_numbers=_CONTRACT0,
                preferred_element_type=jnp.float32,
            ).astype(jnp.bfloat16)
            r = pltpu.make_async_remote_copy(
                src_ref=sendx.at[c],
                dst_ref=xq.at[c],
                send_sem=xs_sems.at[c],
                recv_sem=xr_sems.at[c],
                device_id=xpeer,
                device_id_type=pl.DeviceIdType.MESH,
            )
            r.start()
            x_rdmas.append(r)

        def _accum(row_block, buf, c):
            if row_block == 0:
                out_ref[0:Q, pl.ds(c * FC, FC)] += buf[c, :, :].astype(
                    jnp.float32
                )
            else:
                out_ref[Q : 2 * Q, pl.ds(c * FC, FC)] += buf[c, :, :].astype(
                    jnp.float32
                )

        y_rdmas = []
        for c in range(NC):
            x_rdmas[c].wait_recv()
            r = pltpu.make_async_remote_copy(
                src_ref=xq.at[c],
                dst_ref=yq.at[c],
                send_sem=ys_sems.at[c],
                recv_sem=yr_sems.at[c],
                device_id=ypeer,
                device_id_type=pl.DeviceIdType.MESH,
            )
            r.start()
            y_rdmas.append(r)
            out_ref[:, pl.ds(c * FC, FC)] = lax.dot_general(
                x_ref[:, pl.ds(mx * HALF, HALF)],
                dy_ref[:, pl.ds(c * FC, FC)],
                dimension_numbers=_CONTRACT0,
                preferred_element_type=jnp.float32,
            )
            pl.when(my == 0)(lambda c=c: _accum(0, xq, c))
            pl.when(my == 1)(lambda c=c: _accum(1, xq, c))

        for c in range(NC):
            y_rdmas[c].wait_recv()
            pl.when(my == 0)(lambda c=c: _accum(1, yq, c))
            pl.when(my == 1)(lambda c=c: _accum(0, yq, c))

        for c in range(NC):
            x_rdmas[c].wait_send()
            y_rdmas[c].wait_send()

    return pl.pallas_call(
        body,
        out_shape=jax.ShapeDtypeStruct((HALF, F), jnp.float32),
        in_specs=[
            pl.BlockSpec(memory_space=pltpu.VMEM),
            pl.BlockSpec(memory_space=pltpu.VMEM),
        ],
        out_specs=pl.BlockSpec(memory_space=pltpu.VMEM),
        scratch_shapes=[
            pltpu.VMEM((NC, Q, FC), jnp.bfloat16),
            pltpu.VMEM((NC, Q, FC), jnp.bfloat16),
            pltpu.VMEM((NC, Q, FC), jnp.bfloat16),
            pltpu.SemaphoreType.DMA((NC,)),
            pltpu.SemaphoreType.DMA((NC,)),
            pltpu.SemaphoreType.DMA((NC,)),
            pltpu.SemaphoreType.DMA((NC,)),
        ],
        compiler_params=pltpu.CompilerParams(
            collective_id=0,
            vmem_limit_bytes=100 * 1024 * 1024,
        ),
    )(x, dy)
